# baseline (speedup 1.0000x reference)
import jax
import jax.numpy as jnp
from jax import lax
from jax.experimental import pallas as pl
from jax.experimental.pallas import tpu as pltpu
from jax.experimental.pallas import tpu_sc as plsc

BS = 8
NQ = 2000
NQP = 2048
M = 400
MP = 512
NC = 2
NS = 16
NW = NC * NS
ROWS_PER_W = 2 * BS * NQ // NW
CH = 200
NCHUNK = ROWS_PER_W // CH
MV = M // 16


def _cvt_body(pre_ref, anch_ref, tgt_ref, box_ref, tgt_out_ref):
    i = pl.program_id(0)
    b = jnp.where(i < BS, pre_ref[0], anch_ref[0])
    bt = b.T
    x0, y0, x1, y1 = bt[0:1], bt[1:2], bt[2:3], bt[3:4]
    cvt = jnp.concatenate(
        [(x0 + x1) * 0.5, (y0 + y1) * 0.5, x1 - x0, y1 - y0], axis=0
    )
    box_ref[0, 0] = jnp.pad(cvt, ((0, 4), (0, NQP - NQ)))
    tt = tgt_ref[...]
    tx0, ty0, tx1, ty1 = tt[0:1], tt[1:2], tt[2:3], tt[3:4]
    tcvt = jnp.concatenate(
        [(tx0 + tx1) * 0.5, (ty0 + ty1) * 0.5, tx1 - tx0, ty1 - ty0], axis=0
    )
    tgt_out_ref[...] = jnp.pad(tcvt, ((0, 4), (0, MP - M)))


def _convert_planar(pre_boxes, anchors, targets):
    return pl.pallas_call(
        _cvt_body,
        grid=(2 * BS,),
        in_specs=[
            pl.BlockSpec((1, NQ, 4), lambda i: (i % BS, 0, 0)),
            pl.BlockSpec((1, NQ, 4), lambda i: (i % BS, 0, 0)),
            pl.BlockSpec((4, M), lambda i: (0, 0)),
        ],
        out_specs=[
            pl.BlockSpec((1, 1, 8, NQP), lambda i: (i // BS, i % BS, 0, 0)),
            pl.BlockSpec((8, MP), lambda i: (0, 0)),
        ],
        out_shape=[
            jax.ShapeDtypeStruct((2, BS, 8, NQP), jnp.float32),
            jax.ShapeDtypeStruct((8, MP), jnp.float32),
        ],
    )(pre_boxes, anchors, targets.T)


def _sc_body(box_hbm, tgt_hbm, out_hbm, bxy_v, tv_v, outbuf_v):
    wid = lax.axis_index("s") * NC + lax.axis_index("c")
    a = wid // 16
    b = (wid % 16) // 2
    h = wid % 2
    row0 = h * (NQ // 2)

    pltpu.sync_copy(tgt_hbm, tv_v)
    pltpu.sync_copy(box_hbm.at[a, b], bxy_v)

    def chunk_body(c, carry):
        def grp_body(i8, carry2):
            off = row0 + c * CH + i8 * 8
            bcx16 = bxy_v[0, pl.ds(off, 16)]
            bcy16 = bxy_v[1, pl.ds(off, 16)]
            bw16 = bxy_v[2, pl.ds(off, 16)]
            bh16 = bxy_v[3, pl.ds(off, 16)]
            bcx = [jnp.full((16,), bcx16[r]) for r in range(8)]
            bcy = [jnp.full((16,), bcy16[r]) for r in range(8)]
            bw = [jnp.full((16,), bw16[r]) for r in range(8)]
            bh = [jnp.full((16,), bh16[r]) for r in range(8)]
            for jv in range(MV):
                tcx = tv_v[0, pl.ds(jv * 16, 16)]
                tcy = tv_v[1, pl.ds(jv * 16, 16)]
                tw = tv_v[2, pl.ds(jv * 16, 16)]
                th = tv_v[3, pl.ds(jv * 16, 16)]
                for r in range(8):
                    cost = (
                        jnp.abs(bcx[r] - tcx)
                        + jnp.abs(bcy[r] - tcy)
                        + jnp.abs(bw[r] - tw)
                        + jnp.abs(bh[r] - th)
                    )
                    outbuf_v[i8 * 8 + r, pl.ds(jv * 16, 16)] = cost
            return carry2

        lax.fori_loop(0, CH // 8, grp_body, 0)
        pltpu.sync_copy(
            outbuf_v,
            out_hbm.at[a, b, pl.ds(row0 + c * CH, CH)],
        )
        return carry

    lax.fori_loop(0, NCHUNK, chunk_body, 0)


def kernel(pre_boxes, anchors, targets):
    box_planar, tgt_planar = _convert_planar(pre_boxes, anchors, targets)
    mesh = plsc.VectorSubcoreMesh(core_axis_name="c", subcore_axis_name="s")
    f = pl.kernel(
        _sc_body,
        out_type=jax.ShapeDtypeStruct((2, BS, NQ, M), jnp.float32),
        mesh=mesh,
        scratch_types=[
            pltpu.VMEM((8, NQP), jnp.float32),
            pltpu.VMEM((8, MP), jnp.float32),
            pltpu.VMEM((CH, M), jnp.float32),
        ],
        compiler_params=pltpu.CompilerParams(
            needs_layout_passes=False, use_tc_tiling_on_sc=True
        ),
    )
    return f(box_planar, tgt_planar)


# speedup vs baseline: 1.1111x; 1.1111x over previous
import jax
import jax.numpy as jnp
from jax import lax
from jax.experimental import pallas as pl
from jax.experimental.pallas import tpu as pltpu
from jax.experimental.pallas import tpu_sc as plsc

BS = 8
NQ = 2000
NQP = 2048
M = 400
MP = 512
NC = 2
NS = 16
NW = NC * NS
ROWS_PER_W = 2 * BS * NQ // NW
CH = 200
NCHUNK = ROWS_PER_W // CH
MV = M // 16


def _cvt_body(pre_ref, anch_ref, tgt_ref, box_ref, tgt_out_ref):
    i = pl.program_id(0)
    bt = jnp.where(i < BS, pre_ref[0], anch_ref[0])
    x0, y0, x1, y1 = bt[0:1], bt[1:2], bt[2:3], bt[3:4]
    cvt = jnp.concatenate(
        [(x0 + x1) * 0.5, (y0 + y1) * 0.5, x1 - x0, y1 - y0], axis=0
    )
    box_ref[0, 0] = jnp.pad(cvt, ((0, 4), (0, NQP - NQ)))
    tt = tgt_ref[...]
    tx0, ty0, tx1, ty1 = tt[0:1], tt[1:2], tt[2:3], tt[3:4]
    tcvt = jnp.concatenate(
        [(tx0 + tx1) * 0.5, (ty0 + ty1) * 0.5, tx1 - tx0, ty1 - ty0], axis=0
    )
    tgt_out_ref[...] = jnp.pad(tcvt, ((0, 4), (0, MP - M)))


def _convert_planar(pre_boxes, anchors, targets):
    return pl.pallas_call(
        _cvt_body,
        grid=(2 * BS,),
        in_specs=[
            pl.BlockSpec((1, 4, NQ), lambda i: (i % BS, 0, 0)),
            pl.BlockSpec((1, 4, NQ), lambda i: (i % BS, 0, 0)),
            pl.BlockSpec((4, M), lambda i: (0, 0)),
        ],
        out_specs=[
            pl.BlockSpec((1, 1, 8, NQP), lambda i: (i // BS, i % BS, 0, 0)),
            pl.BlockSpec((8, MP), lambda i: (0, 0)),
        ],
        out_shape=[
            jax.ShapeDtypeStruct((2, BS, 8, NQP), jnp.float32),
            jax.ShapeDtypeStruct((8, MP), jnp.float32),
        ],
    )(jnp.swapaxes(pre_boxes, 1, 2), jnp.swapaxes(anchors, 1, 2), targets.T)


def _sc_body(box_hbm, tgt_hbm, out_hbm, bxy_v, tv_v, outbuf_v):
    wid = lax.axis_index("s") * NC + lax.axis_index("c")
    a = wid // 16
    b = (wid % 16) // 2
    h = wid % 2
    row0 = h * (NQ // 2)

    pltpu.sync_copy(tgt_hbm, tv_v)
    pltpu.sync_copy(box_hbm.at[a, b], bxy_v)

    def chunk_body(c, carry):
        def grp_body(i8, carry2):
            off = row0 + c * CH + i8 * 8
            bcx16 = bxy_v[0, pl.ds(off, 16)]
            bcy16 = bxy_v[1, pl.ds(off, 16)]
            bw16 = bxy_v[2, pl.ds(off, 16)]
            bh16 = bxy_v[3, pl.ds(off, 16)]
            bcx = [jnp.full((16,), bcx16[r]) for r in range(8)]
            bcy = [jnp.full((16,), bcy16[r]) for r in range(8)]
            bw = [jnp.full((16,), bw16[r]) for r in range(8)]
            bh = [jnp.full((16,), bh16[r]) for r in range(8)]
            for jv in range(MV):
                tcx = tv_v[0, pl.ds(jv * 16, 16)]
                tcy = tv_v[1, pl.ds(jv * 16, 16)]
                tw = tv_v[2, pl.ds(jv * 16, 16)]
                th = tv_v[3, pl.ds(jv * 16, 16)]
                for r in range(8):
                    cost = (
                        jnp.abs(bcx[r] - tcx)
                        + jnp.abs(bcy[r] - tcy)
                        + jnp.abs(bw[r] - tw)
                        + jnp.abs(bh[r] - th)
                    )
                    outbuf_v[i8 * 8 + r, pl.ds(jv * 16, 16)] = cost
            return carry2

        lax.fori_loop(0, CH // 8, grp_body, 0)
        pltpu.sync_copy(
            outbuf_v,
            out_hbm.at[a, b, pl.ds(row0 + c * CH, CH)],
        )
        return carry

    lax.fori_loop(0, NCHUNK, chunk_body, 0)


def kernel(pre_boxes, anchors, targets):
    box_planar, tgt_planar = _convert_planar(pre_boxes, anchors, targets)
    mesh = plsc.VectorSubcoreMesh(core_axis_name="c", subcore_axis_name="s")
    f = pl.kernel(
        _sc_body,
        out_type=jax.ShapeDtypeStruct((2, BS, NQ, M), jnp.float32),
        mesh=mesh,
        scratch_types=[
            pltpu.VMEM((8, NQP), jnp.float32),
            pltpu.VMEM((8, MP), jnp.float32),
            pltpu.VMEM((CH, M), jnp.float32),
        ],
        compiler_params=pltpu.CompilerParams(
            needs_layout_passes=False, use_tc_tiling_on_sc=True
        ),
    )
    return f(box_planar, tgt_planar)
